# baseline (speedup 1.0000x reference)
import functools

import jax
import jax.numpy as jnp
from jax import lax
from jax.experimental import pallas as pl
from jax.experimental.pallas import tpu as pltpu
from jax.experimental.pallas import tpu_sc as plsc

_NC = 2
_NS = 16
_L = 16
_NW = _NC * _NS

_B = 16384
_D = 64
_V = 1000000
_BPW = _B // _NW
_CHUNK = 128
_GROUPS = _BPW // _L
_BLK = 16384


def _fold_body(w1_ref, b1_ref, w2_ref, b2_ref, w3_ref, b3_ref,
               weff_ref, beff_ref):
    w2v = w2_ref[...]
    w3v = w3_ref[...]
    w23 = jnp.dot(w2v, w3v, preferred_element_type=jnp.float32)
    weff_ref[...] = jnp.dot(w1_ref[...], w23,
                            preferred_element_type=jnp.float32)
    beff_ref[...] = (jnp.dot(b1_ref[...], w23,
                             preferred_element_type=jnp.float32)
                     + jnp.dot(b2_ref[...], w3v,
                               preferred_element_type=jnp.float32)
                     + b3_ref[...])


_fold = pl.pallas_call(
    _fold_body,
    out_shape=(jax.ShapeDtypeStruct((2 * _D, 1), jnp.float32),
               jax.ShapeDtypeStruct((1, 1), jnp.float32)),
)


def _proj_body(w_ref, t_ref, c_ref, pt_ref, pc_ref):
    w = w_ref[...]
    wt = w[:, :_D]
    wc = w[:, _D:]
    pt_ref[...] = jnp.dot(wt, t_ref[...],
                          preferred_element_type=jnp.float32)[0]
    pc_ref[...] = jnp.dot(wc, c_ref[...],
                          preferred_element_type=jnp.float32)[0]


_proj = pl.pallas_call(
    _proj_body,
    grid=(pl.cdiv(_V, _BLK),),
    in_specs=[
        pl.BlockSpec((1, 2 * _D), lambda i: (0, 0)),
        pl.BlockSpec((_D, _BLK), lambda i: (0, i)),
        pl.BlockSpec((_D, _BLK), lambda i: (0, i)),
    ],
    out_specs=[
        pl.BlockSpec((_BLK,), lambda i: (i,)),
        pl.BlockSpec((_BLK,), lambda i: (i,)),
    ],
    out_shape=(jax.ShapeDtypeStruct((_V,), jnp.float32),
               jax.ShapeDtypeStruct((_V,), jnp.float32)),
)


_PROBE_CW = 512
_PROBE_CHUNKS = 32


def _probe_body(tt_hbm, out_hbm, buf_a, buf_b, out_v, sem_a, sem_b):
    wid = lax.axis_index("s") * _NC + lax.axis_index("c")
    span = _PROBE_CHUNKS * _PROBE_CW
    base = wid * span
    bufs = (buf_a, buf_b)
    sems = (sem_a, sem_b)
    cps = [None, None]
    cps[0] = pltpu.async_copy(tt_hbm.at[:, pl.ds(base, _PROBE_CW)],
                              buf_a, sem_a)
    for k in range(1, _PROBE_CHUNKS):
        cps[k % 2] = pltpu.async_copy(
            tt_hbm.at[:, pl.ds(base + k * _PROBE_CW, _PROBE_CW)],
            bufs[k % 2], sems[k % 2])
        cps[(k - 1) % 2].wait()
    cps[(_PROBE_CHUNKS - 1) % 2].wait()
    out_v[...] = buf_a[0, pl.ds(0, _L)]
    pltpu.sync_copy(out_v, out_hbm.at[pl.ds(wid * _L, _L)])


_sc_probe = functools.partial(
    pl.kernel,
    mesh=plsc.VectorSubcoreMesh(core_axis_name="c", subcore_axis_name="s"),
    out_type=jax.ShapeDtypeStruct((_NW * _L,), jnp.float32),
    compiler_params=pltpu.CompilerParams(needs_layout_passes=False,
                                         use_tc_tiling_on_sc=True),
    scratch_types=[
        pltpu.VMEM((_D, _PROBE_CW), jnp.float32),
        pltpu.VMEM((_D, _PROBE_CW), jnp.float32),
        pltpu.VMEM((_L,), jnp.float32),
        pltpu.SemaphoreType.DMA,
        pltpu.SemaphoreType.DMA,
    ],
)(_probe_body)


def _sc_body(tidx_hbm, cidx_hbm, pt_hbm, pc_hbm, b_hbm, probe_hbm,
             out_hbm,
             tidx_v, cidx_v, gt_v, gc_v, b_v, out_v, sem):
    del probe_hbm
    wid = lax.axis_index("s") * _NC + lax.axis_index("c")
    base = wid * _BPW

    pltpu.sync_copy(tidx_hbm.at[pl.ds(base, _BPW)], tidx_v)
    pltpu.sync_copy(cidx_hbm.at[pl.ds(base, _BPW)], cidx_v)
    copies = []
    for j in range(_BPW // _CHUNK):
        sl = pl.ds(j * _CHUNK, _CHUNK)
        copies.append(pltpu.async_copy(pt_hbm.at[tidx_v.at[sl]],
                                       gt_v.at[sl], sem))
        copies.append(pltpu.async_copy(pc_hbm.at[cidx_v.at[sl]],
                                       gc_v.at[sl], sem))
    pltpu.sync_copy(b_hbm, b_v)
    for c in copies:
        c.wait()

    bvec = b_v[...]

    def group(g, carry):
        sl = pl.ds(g * _L, _L)
        x = gt_v[sl] + gc_v[sl] + bvec
        out_v[sl] = 1.0 / (1.0 + jnp.exp(-x))
        return carry

    lax.fori_loop(0, _GROUPS, group, 0)
    pltpu.sync_copy(out_v, out_hbm.at[pl.ds(base, _BPW)])


_sc_lookup = functools.partial(
    pl.kernel,
    mesh=plsc.VectorSubcoreMesh(core_axis_name="c", subcore_axis_name="s"),
    out_type=jax.ShapeDtypeStruct((_B,), jnp.float32),
    compiler_params=pltpu.CompilerParams(needs_layout_passes=False,
                                         use_tc_tiling_on_sc=False),
    scratch_types=[
        pltpu.VMEM((_BPW,), jnp.int32),
        pltpu.VMEM((_BPW,), jnp.int32),
        pltpu.VMEM((_BPW,), jnp.float32),
        pltpu.VMEM((_BPW,), jnp.float32),
        pltpu.VMEM((_L,), jnp.float32),
        pltpu.VMEM((_BPW,), jnp.float32),
        pltpu.SemaphoreType.DMA,
    ],
)(_sc_body)


def kernel(inputs, target_table, context_table, W1, b1, W2, b2, W3, b3):
    tgt = inputs[:, 0]
    ctx = inputs[:, 1]
    weff, beff = _fold(W1, b1.reshape(1, -1), W2, b2.reshape(1, -1),
                       W3, b3.reshape(1, 1))
    proj_t, proj_c = _proj(weff.reshape(1, -1),
                           target_table.T, context_table.T)
    bsplat = jnp.tile(beff.reshape(-1), _L)
    probe = _sc_probe(target_table.T)
    out = _sc_lookup(tgt, ctx, proj_t, proj_c, bsplat, probe)
    return out.reshape(_B, 1)


# speedup vs baseline: 1.2215x; 1.2215x over previous
import functools

import jax
import jax.numpy as jnp
from jax import lax
from jax.experimental import pallas as pl
from jax.experimental.pallas import tpu as pltpu
from jax.experimental.pallas import tpu_sc as plsc

_NC = 2
_NS = 16
_L = 16
_NW = _NC * _NS

_B = 16384
_D = 64
_V = 1000000
_BPW = _B // _NW
_CHUNK = 128
_GROUPS = _BPW // _L
_BLK = 16384


def _proj_body(w1_ref, w2_ref, w3_ref, t_ref, c_ref, pt_ref, pc_ref):
    w23 = jnp.dot(w2_ref[...], w3_ref[...],
                  preferred_element_type=jnp.float32)
    weff = jnp.dot(w1_ref[...], w23,
                   preferred_element_type=jnp.float32)
    wt = weff[:_D, :].T
    wc = weff[_D:, :].T
    pt_ref[...] = jnp.dot(wt, t_ref[...],
                          preferred_element_type=jnp.float32)[0]
    pc_ref[...] = jnp.dot(wc, c_ref[...],
                          preferred_element_type=jnp.float32)[0]


_proj = pl.pallas_call(
    _proj_body,
    grid=(pl.cdiv(_V, _BLK),),
    in_specs=[
        pl.BlockSpec((2 * _D, _D), lambda i: (0, 0)),
        pl.BlockSpec((_D, _D), lambda i: (0, 0)),
        pl.BlockSpec((_D, 1), lambda i: (0, 0)),
        pl.BlockSpec((_D, _BLK), lambda i: (0, i)),
        pl.BlockSpec((_D, _BLK), lambda i: (0, i)),
    ],
    out_specs=[
        pl.BlockSpec((_BLK,), lambda i: (i,)),
        pl.BlockSpec((_BLK,), lambda i: (i,)),
    ],
    out_shape=(jax.ShapeDtypeStruct((_V,), jnp.float32),
               jax.ShapeDtypeStruct((_V,), jnp.float32)),
)


def _sc_body(tidx_hbm, cidx_hbm, pt_hbm, pc_hbm, b1_hbm, b2_hbm, b3_hbm,
             w2_hbm, w3_hbm,
             out_hbm,
             tidx_v, cidx_v, gt_v, gc_v, b1_v, b2_v, b3_v, w2_v, w3_v,
             out_v, sem):
    wid = lax.axis_index("s") * _NC + lax.axis_index("c")
    base = wid * _BPW

    pltpu.sync_copy(tidx_hbm.at[pl.ds(base, _BPW)], tidx_v)
    pltpu.sync_copy(cidx_hbm.at[pl.ds(base, _BPW)], cidx_v)
    copies = []
    for j in range(_BPW // _CHUNK):
        sl = pl.ds(j * _CHUNK, _CHUNK)
        copies.append(pltpu.async_copy(pt_hbm.at[tidx_v.at[sl]],
                                       gt_v.at[sl], sem))
        copies.append(pltpu.async_copy(pc_hbm.at[cidx_v.at[sl]],
                                       gc_v.at[sl], sem))
    pltpu.sync_copy(b1_hbm, b1_v)
    pltpu.sync_copy(b2_hbm, b2_v)
    pltpu.sync_copy(b3_hbm, b3_v)
    pltpu.sync_copy(w2_hbm, w2_v)
    pltpu.sync_copy(w3_hbm, w3_v)

    nv = _D // _L
    uvecs = []
    for jb in range(nv):
        acc = jnp.zeros((_L,), jnp.float32)
        for i in range(_D):
            b1i = b1_v[pl.ds((i // _L) * _L, _L)][i % _L]
            acc = acc + w2_v[i, pl.ds(jb * _L, _L)] * b1i
        uvecs.append(acc + b2_v[pl.ds(jb * _L, _L)])
    acc = jnp.zeros((_L,), jnp.float32)
    for jb in range(nv):
        acc = acc + uvecs[jb] * w3_v[pl.ds(jb * _L, _L)]
    beff = jnp.sum(acc) + b3_v[pl.ds(0, _L)][0]

    for c in copies:
        c.wait()

    def group(g, carry):
        sl = pl.ds(g * _L, _L)
        x = gt_v[sl] + gc_v[sl] + beff
        out_v[sl] = 1.0 / (1.0 + jnp.exp(-x))
        return carry

    lax.fori_loop(0, _GROUPS, group, 0)
    pltpu.sync_copy(out_v, out_hbm.at[pl.ds(base, _BPW)])


_sc_lookup = functools.partial(
    pl.kernel,
    mesh=plsc.VectorSubcoreMesh(core_axis_name="c", subcore_axis_name="s"),
    out_type=jax.ShapeDtypeStruct((_B,), jnp.float32),
    compiler_params=pltpu.CompilerParams(needs_layout_passes=False,
                                         use_tc_tiling_on_sc=False),
    scratch_types=[
        pltpu.VMEM((_BPW,), jnp.int32),
        pltpu.VMEM((_BPW,), jnp.int32),
        pltpu.VMEM((_BPW,), jnp.float32),
        pltpu.VMEM((_BPW,), jnp.float32),
        pltpu.VMEM((_D,), jnp.float32),
        pltpu.VMEM((_D,), jnp.float32),
        pltpu.VMEM((_L,), jnp.float32),
        pltpu.VMEM((_D, _D), jnp.float32),
        pltpu.VMEM((_D,), jnp.float32),
        pltpu.VMEM((_BPW,), jnp.float32),
        pltpu.SemaphoreType.DMA,
    ],
)(_sc_body)


def kernel(inputs, target_table, context_table, W1, b1, W2, b2, W3, b3):
    tgt = inputs[:, 0]
    ctx = inputs[:, 1]
    proj_t, proj_c = _proj(W1, W2, W3, target_table.T, context_table.T)
    b3pad = jnp.pad(b3, (0, _L - 1))
    out = _sc_lookup(tgt, ctx, proj_t, proj_c, b1, b2, b3pad, W2,
                     W3.reshape(-1))
    return out.reshape(_B, 1)


# speedup vs baseline: 1.2243x; 1.0023x over previous
import functools

import jax
import jax.numpy as jnp
from jax import lax
from jax.experimental import pallas as pl
from jax.experimental.pallas import tpu as pltpu
from jax.experimental.pallas import tpu_sc as plsc

_NC = 2
_NS = 16
_L = 16
_NW = _NC * _NS

_B = 16384
_D = 64
_V = 1000000
_BPW = _B // _NW
_CHUNK = 128
_GROUPS = _BPW // _L
_BLK = 16384


def _fold_body(w1_ref, b1_ref, w2_ref, b2_ref, w3_ref, b3_ref,
               weff_ref, beff_ref):
    w2v = w2_ref[...]
    w3v = w3_ref[...]
    w23 = jnp.dot(w2v, w3v, preferred_element_type=jnp.float32)
    weff_ref[...] = jnp.dot(w1_ref[...], w23,
                            preferred_element_type=jnp.float32)
    beff_ref[...] = (jnp.dot(b1_ref[...], w23,
                             preferred_element_type=jnp.float32)
                     + jnp.dot(b2_ref[...], w3v,
                               preferred_element_type=jnp.float32)
                     + b3_ref[...])


_fold = pl.pallas_call(
    _fold_body,
    out_shape=(jax.ShapeDtypeStruct((2 * _D, 1), jnp.float32),
               jax.ShapeDtypeStruct((1, 1), jnp.float32)),
)


def _proj_body(w_ref, t_ref, c_ref, pt_ref, pc_ref):
    w = w_ref[...]
    wt = w[:, :_D]
    wc = w[:, _D:]
    pt_ref[...] = jnp.dot(wt, t_ref[...],
                          preferred_element_type=jnp.float32)[0]
    pc_ref[...] = jnp.dot(wc, c_ref[...],
                          preferred_element_type=jnp.float32)[0]


_proj = pl.pallas_call(
    _proj_body,
    grid=(pl.cdiv(_V, _BLK),),
    in_specs=[
        pl.BlockSpec((1, 2 * _D), lambda i: (0, 0)),
        pl.BlockSpec((_D, _BLK), lambda i: (0, i)),
        pl.BlockSpec((_D, _BLK), lambda i: (0, i)),
    ],
    out_specs=[
        pl.BlockSpec((_BLK,), lambda i: (i,)),
        pl.BlockSpec((_BLK,), lambda i: (i,)),
    ],
    out_shape=(jax.ShapeDtypeStruct((_V,), jnp.float32),
               jax.ShapeDtypeStruct((_V,), jnp.float32)),
)


def _sc_body(tidx_hbm, cidx_hbm, pt_hbm, pc_hbm, b_hbm,
             out_hbm,
             tidx_v, cidx_v, gt_v, gc_v, b_v, out_v, sem):
    wid = lax.axis_index("s") * _NC + lax.axis_index("c")
    base = wid * _BPW

    pltpu.sync_copy(tidx_hbm.at[pl.ds(base, _BPW)], tidx_v)
    pltpu.sync_copy(cidx_hbm.at[pl.ds(base, _BPW)], cidx_v)
    copies = []
    for j in range(_BPW // _CHUNK):
        sl = pl.ds(j * _CHUNK, _CHUNK)
        copies.append(pltpu.async_copy(pt_hbm.at[tidx_v.at[sl]],
                                       gt_v.at[sl], sem))
        copies.append(pltpu.async_copy(pc_hbm.at[cidx_v.at[sl]],
                                       gc_v.at[sl], sem))
    pltpu.sync_copy(b_hbm, b_v)
    for c in copies:
        c.wait()

    bvec = b_v[...]

    def group(g, carry):
        sl = pl.ds(g * _L, _L)
        x = gt_v[sl] + gc_v[sl] + bvec
        out_v[sl] = 1.0 / (1.0 + jnp.exp(-x))
        return carry

    lax.fori_loop(0, _GROUPS, group, 0)
    pltpu.sync_copy(out_v, out_hbm.at[pl.ds(base, _BPW)])


_sc_lookup = functools.partial(
    pl.kernel,
    mesh=plsc.VectorSubcoreMesh(core_axis_name="c", subcore_axis_name="s"),
    out_type=jax.ShapeDtypeStruct((_B,), jnp.float32),
    compiler_params=pltpu.CompilerParams(needs_layout_passes=False,
                                         use_tc_tiling_on_sc=False),
    scratch_types=[
        pltpu.VMEM((_BPW,), jnp.int32),
        pltpu.VMEM((_BPW,), jnp.int32),
        pltpu.VMEM((_BPW,), jnp.float32),
        pltpu.VMEM((_BPW,), jnp.float32),
        pltpu.VMEM((_L,), jnp.float32),
        pltpu.VMEM((_BPW,), jnp.float32),
        pltpu.SemaphoreType.DMA,
    ],
)(_sc_body)


def kernel(inputs, target_table, context_table, W1, b1, W2, b2, W3, b3):
    tgt = inputs[:, 0]
    ctx = inputs[:, 1]
    weff, beff = _fold(W1, b1.reshape(1, -1), W2, b2.reshape(1, -1),
                       W3, b3.reshape(1, 1))
    proj_t, proj_c = _proj(weff.reshape(1, -1),
                           target_table.T, context_table.T)
    bsplat = jnp.tile(beff.reshape(-1), _L)
    out = _sc_lookup(tgt, ctx, proj_t, proj_c, bsplat)
    return out.reshape(_B, 1)


# speedup vs baseline: 1.2308x; 1.0053x over previous
import functools

import jax
import jax.numpy as jnp
from jax import lax
from jax.experimental import pallas as pl
from jax.experimental.pallas import tpu as pltpu
from jax.experimental.pallas import tpu_sc as plsc

_NC = 2
_NS = 16
_L = 16
_NW = _NC * _NS

_B = 16384
_D = 64
_V = 1000000
_BPW = _B // _NW
_CHUNK = 128
_GROUPS = _BPW // _L
_BLK = 16384


def _fold_body(w1_ref, b1_ref, w2_ref, b2_ref, w3_ref, b3_ref,
               weff_ref, beff_ref):
    w2v = w2_ref[...]
    w3v = w3_ref[...]
    w23 = jnp.dot(w2v, w3v, preferred_element_type=jnp.float32)
    weff_ref[...] = jnp.dot(w1_ref[...], w23,
                            preferred_element_type=jnp.float32)
    beff_ref[...] = (jnp.dot(b1_ref[...], w23,
                             preferred_element_type=jnp.float32)
                     + jnp.dot(b2_ref[...], w3v,
                               preferred_element_type=jnp.float32)
                     + b3_ref[...])


_fold = pl.pallas_call(
    _fold_body,
    out_shape=(jax.ShapeDtypeStruct((2 * _D, 1), jnp.float32),
               jax.ShapeDtypeStruct((1, 1), jnp.float32)),
)


def _proj_body(w_ref, t_ref, c_ref, pt_ref, pc_ref):
    w = w_ref[...]
    wt = w[:, :_D]
    wc = w[:, _D:]
    pt_ref[...] = jnp.dot(wt, t_ref[...],
                          preferred_element_type=jnp.float32)[0]
    pc_ref[...] = jnp.dot(wc, c_ref[...],
                          preferred_element_type=jnp.float32)[0]


_proj = pl.pallas_call(
    _proj_body,
    grid=(pl.cdiv(_V, _BLK),),
    in_specs=[
        pl.BlockSpec((1, 2 * _D), lambda i: (0, 0)),
        pl.BlockSpec((_D, _BLK), lambda i: (0, i)),
        pl.BlockSpec((_D, _BLK), lambda i: (0, i)),
    ],
    out_specs=[
        pl.BlockSpec((_BLK,), lambda i: (i,)),
        pl.BlockSpec((_BLK,), lambda i: (i,)),
    ],
    out_shape=(jax.ShapeDtypeStruct((_V,), jnp.float32),
               jax.ShapeDtypeStruct((_V,), jnp.float32)),
)


def _sc_body(tidx_hbm, cidx_hbm, pt_hbm, pc_hbm, b_hbm,
             out_hbm,
             tidx_v, cidx_v, gt_v, gc_v, b_v, out_v, sem):
    wid = lax.axis_index("s") * _NC + lax.axis_index("c")
    base = wid * _BPW

    ti_cp = pltpu.async_copy(tidx_hbm.at[pl.ds(base, _BPW)], tidx_v, sem)
    ci_cp = pltpu.async_copy(cidx_hbm.at[pl.ds(base, _BPW)], cidx_v, sem)
    ti_cp.wait()
    ci_cp.wait()
    copies = []
    for j in range(_BPW // _CHUNK):
        sl = pl.ds(j * _CHUNK, _CHUNK)
        copies.append(pltpu.async_copy(pt_hbm.at[tidx_v.at[sl]],
                                       gt_v.at[sl], sem))
        copies.append(pltpu.async_copy(pc_hbm.at[cidx_v.at[sl]],
                                       gc_v.at[sl], sem))
    pltpu.sync_copy(b_hbm, b_v)
    for c in copies:
        c.wait()

    bvec = b_v[...]

    def group(g, carry):
        sl = pl.ds(g * _L, _L)
        x = gt_v[sl] + gc_v[sl] + bvec
        out_v[sl] = 1.0 / (1.0 + jnp.exp(-x))
        return carry

    lax.fori_loop(0, _GROUPS, group, 0)
    pltpu.sync_copy(out_v, out_hbm.at[pl.ds(base, _BPW)])


_sc_lookup = functools.partial(
    pl.kernel,
    mesh=plsc.VectorSubcoreMesh(core_axis_name="c", subcore_axis_name="s"),
    out_type=jax.ShapeDtypeStruct((_B,), jnp.float32),
    compiler_params=pltpu.CompilerParams(needs_layout_passes=False,
                                         use_tc_tiling_on_sc=False),
    scratch_types=[
        pltpu.VMEM((_BPW,), jnp.int32),
        pltpu.VMEM((_BPW,), jnp.int32),
        pltpu.VMEM((_BPW,), jnp.float32),
        pltpu.VMEM((_BPW,), jnp.float32),
        pltpu.VMEM((_L,), jnp.float32),
        pltpu.VMEM((_BPW,), jnp.float32),
        pltpu.SemaphoreType.DMA,
    ],
)(_sc_body)


def kernel(inputs, target_table, context_table, W1, b1, W2, b2, W3, b3):
    tgt = inputs[:, 0]
    ctx = inputs[:, 1]
    weff, beff = _fold(W1, b1.reshape(1, -1), W2, b2.reshape(1, -1),
                       W3, b3.reshape(1, 1))
    proj_t, proj_c = _proj(weff.reshape(1, -1),
                           target_table.T, context_table.T)
    bsplat = jnp.tile(beff.reshape(-1), _L)
    out = _sc_lookup(tgt, ctx, proj_t, proj_c, bsplat)
    return out.reshape(_B, 1)


# speedup vs baseline: 1.2318x; 1.0008x over previous
import functools

import jax
import jax.numpy as jnp
from jax import lax
from jax.experimental import pallas as pl
from jax.experimental.pallas import tpu as pltpu
from jax.experimental.pallas import tpu_sc as plsc

_NC = 2
_NS = 16
_L = 16
_NW = _NC * _NS

_B = 16384
_D = 64
_V = 1000000
_BPW = _B // _NW
_CHUNK = 128
_GROUPS = _BPW // _L
_BLK = 20480


def _fold_body(w1_ref, b1_ref, w2_ref, b2_ref, w3_ref, b3_ref,
               weff_ref, beff_ref):
    w2v = w2_ref[...]
    w3v = w3_ref[...]
    w23 = jnp.dot(w2v, w3v, preferred_element_type=jnp.float32)
    weff_ref[...] = jnp.dot(w1_ref[...], w23,
                            preferred_element_type=jnp.float32)
    beff_ref[...] = (jnp.dot(b1_ref[...], w23,
                             preferred_element_type=jnp.float32)
                     + jnp.dot(b2_ref[...], w3v,
                               preferred_element_type=jnp.float32)
                     + b3_ref[...])


_fold = pl.pallas_call(
    _fold_body,
    out_shape=(jax.ShapeDtypeStruct((2 * _D, 1), jnp.float32),
               jax.ShapeDtypeStruct((1, 1), jnp.float32)),
)


def _proj_body(w_ref, t_ref, c_ref, pt_ref, pc_ref):
    w = w_ref[...]
    wt = w[:, :_D]
    wc = w[:, _D:]
    pt_ref[...] = jnp.dot(wt, t_ref[...],
                          preferred_element_type=jnp.float32)[0]
    pc_ref[...] = jnp.dot(wc, c_ref[...],
                          preferred_element_type=jnp.float32)[0]


_proj = pl.pallas_call(
    _proj_body,
    grid=(pl.cdiv(_V, _BLK),),
    in_specs=[
        pl.BlockSpec((1, 2 * _D), lambda i: (0, 0)),
        pl.BlockSpec((_D, _BLK), lambda i: (0, i)),
        pl.BlockSpec((_D, _BLK), lambda i: (0, i)),
    ],
    out_specs=[
        pl.BlockSpec((_BLK,), lambda i: (i,)),
        pl.BlockSpec((_BLK,), lambda i: (i,)),
    ],
    out_shape=(jax.ShapeDtypeStruct((_V,), jnp.float32),
               jax.ShapeDtypeStruct((_V,), jnp.float32)),
)


def _sc_body(tidx_hbm, cidx_hbm, pt_hbm, pc_hbm, b_hbm,
             out_hbm,
             tidx_v, cidx_v, gt_v, gc_v, b_v, out_v, sem):
    wid = lax.axis_index("s") * _NC + lax.axis_index("c")
    base = wid * _BPW

    ti_cp = pltpu.async_copy(tidx_hbm.at[pl.ds(base, _BPW)], tidx_v, sem)
    ci_cp = pltpu.async_copy(cidx_hbm.at[pl.ds(base, _BPW)], cidx_v, sem)
    ti_cp.wait()
    ci_cp.wait()
    copies = []
    for j in range(_BPW // _CHUNK):
        sl = pl.ds(j * _CHUNK, _CHUNK)
        copies.append(pltpu.async_copy(pt_hbm.at[tidx_v.at[sl]],
                                       gt_v.at[sl], sem))
        copies.append(pltpu.async_copy(pc_hbm.at[cidx_v.at[sl]],
                                       gc_v.at[sl], sem))
    pltpu.sync_copy(b_hbm, b_v)
    for c in copies:
        c.wait()

    bvec = b_v[...]

    def group(g, carry):
        sl = pl.ds(g * _L, _L)
        x = gt_v[sl] + gc_v[sl] + bvec
        out_v[sl] = 1.0 / (1.0 + jnp.exp(-x))
        return carry

    lax.fori_loop(0, _GROUPS, group, 0)
    pltpu.sync_copy(out_v, out_hbm.at[pl.ds(base, _BPW)])


_sc_lookup = functools.partial(
    pl.kernel,
    mesh=plsc.VectorSubcoreMesh(core_axis_name="c", subcore_axis_name="s"),
    out_type=jax.ShapeDtypeStruct((_B,), jnp.float32),
    compiler_params=pltpu.CompilerParams(needs_layout_passes=False,
                                         use_tc_tiling_on_sc=False),
    scratch_types=[
        pltpu.VMEM((_BPW,), jnp.int32),
        pltpu.VMEM((_BPW,), jnp.int32),
        pltpu.VMEM((_BPW,), jnp.float32),
        pltpu.VMEM((_BPW,), jnp.float32),
        pltpu.VMEM((_L,), jnp.float32),
        pltpu.VMEM((_BPW,), jnp.float32),
        pltpu.SemaphoreType.DMA,
    ],
)(_sc_body)


def kernel(inputs, target_table, context_table, W1, b1, W2, b2, W3, b3):
    tgt = inputs[:, 0]
    ctx = inputs[:, 1]
    weff, beff = _fold(W1, b1.reshape(1, -1), W2, b2.reshape(1, -1),
                       W3, b3.reshape(1, 1))
    proj_t, proj_c = _proj(weff.reshape(1, -1),
                           target_table.T, context_table.T)
    bsplat = jnp.tile(beff.reshape(-1), _L)
    out = _sc_lookup(tgt, ctx, proj_t, proj_c, bsplat)
    return out.reshape(_B, 1)
